# Initial kernel scaffold; baseline (speedup 1.0000x reference)
import jax
import jax.numpy as jnp
from jax.experimental import pallas as pl


def kernel(inputs):
    raise NotImplementedError("write your pallas kernel here")


# speedup vs baseline: 5.6358x; 5.6358x over previous
import functools

import jax
import jax.numpy as jnp
from jax.experimental import pallas as pl

_NUM_UNITS = 1024
_SCALE = float(_NUM_UNITS) ** 0.5
_BT = 512


def _pe_tile(o_ref, *, bt, num_units):
    t = pl.program_id(0)
    row = jax.lax.broadcasted_iota(jnp.int32, (bt, num_units), 0)
    col = jax.lax.broadcasted_iota(jnp.int32, (bt, num_units), 1)
    pos = (row + t * bt).astype(jnp.float32)
    i = col.astype(jnp.float32)
    denom = jnp.power(10000.0, 2.0 * i / float(num_units))
    angle = pos / denom
    parity = col % 2
    val = jnp.where(parity == 0, jnp.sin(angle), jnp.cos(angle))
    val = jnp.where(pos == 0.0, 0.0, val) * _SCALE
    o_ref[...] = jnp.broadcast_to(val[None], o_ref.shape)


def kernel(inputs):
    n, t_len = inputs.shape
    num_units = _NUM_UNITS
    bt = _BT
    grid = (t_len // bt,)
    out = pl.pallas_call(
        functools.partial(_pe_tile, bt=bt, num_units=num_units),
        grid=grid,
        out_specs=pl.BlockSpec((n, bt, num_units), lambda g: (0, g, 0)),
        out_shape=jax.ShapeDtypeStruct((n, t_len, num_units), jnp.float32),
    )()
    return out


# speedup vs baseline: 6.1971x; 1.0996x over previous
import functools
import math

import jax
import jax.numpy as jnp
from jax.experimental import pallas as pl

_NUM_UNITS = 1024
_SCALE = float(_NUM_UNITS) ** 0.5
_BT = 512


def _pe_tile(o_ref, *, bt, num_units):
    t = pl.program_id(0)
    col = jax.lax.broadcasted_iota(jnp.int32, (1, num_units), 1)
    inv = jnp.exp(
        col.astype(jnp.float32) * (-2.0 * math.log(10000.0) / float(num_units))
    )
    phase = (col % 2).astype(jnp.float32) * (math.pi / 2.0)
    row = jax.lax.broadcasted_iota(jnp.int32, (bt, num_units), 0)
    pos = (row + t * bt).astype(jnp.float32)
    val = jnp.sin(pos * inv + phase)
    val = jnp.where(pos == 0.0, 0.0, val) * _SCALE
    o_ref[...] = jnp.broadcast_to(val[None], o_ref.shape)


def kernel(inputs):
    n, t_len = inputs.shape
    num_units = _NUM_UNITS
    bt = _BT
    grid = (t_len // bt,)
    out = pl.pallas_call(
        functools.partial(_pe_tile, bt=bt, num_units=num_units),
        grid=grid,
        out_specs=pl.BlockSpec((n, bt, num_units), lambda g: (0, g, 0)),
        out_shape=jax.ShapeDtypeStruct((n, t_len, num_units), jnp.float32),
    )()
    return out


# speedup vs baseline: 12.6966x; 2.0488x over previous
import functools
import math

import jax
import jax.numpy as jnp
from jax.experimental import pallas as pl
from jax.experimental.pallas import tpu as pltpu

_NUM_UNITS = 1024
_SCALE = float(_NUM_UNITS) ** 0.5
_BT = 128


def _pe_tile(o_ref, sinx_ref, cosx_ref, *, bt, num_units):
    t = pl.program_id(0)
    col = jax.lax.broadcasted_iota(jnp.int32, (1, num_units), 1)
    inv = jnp.exp(
        col.astype(jnp.float32) * (-2.0 * math.log(10000.0) / float(num_units))
    )

    @pl.when(t == 0)
    def _():
        phase = (col % 2).astype(jnp.float32) * (math.pi / 2.0)
        r = jax.lax.broadcasted_iota(jnp.int32, (bt, num_units), 0).astype(
            jnp.float32
        )
        x = r * inv + phase
        sinx_ref[...] = jnp.sin(x)
        cosx_ref[...] = jnp.cos(x)

    p = (t * bt).astype(jnp.float32) * inv
    sp = jnp.sin(p) * _SCALE
    cp = jnp.cos(p) * _SCALE
    val = sp * cosx_ref[...] + cp * sinx_ref[...]
    o_ref[...] = jnp.broadcast_to(val[None], o_ref.shape)

    @pl.when(t == 0)
    def _():
        o_ref[:, 0:1, :] = jnp.zeros_like(o_ref[:, 0:1, :])


def kernel(inputs):
    n, t_len = inputs.shape
    num_units = _NUM_UNITS
    bt = _BT
    grid = (t_len // bt,)
    out = pl.pallas_call(
        functools.partial(_pe_tile, bt=bt, num_units=num_units),
        grid=grid,
        out_specs=pl.BlockSpec((n, bt, num_units), lambda g: (0, g, 0)),
        out_shape=jax.ShapeDtypeStruct((n, t_len, num_units), jnp.float32),
        scratch_shapes=[
            pltpu.VMEM((bt, num_units), jnp.float32),
            pltpu.VMEM((bt, num_units), jnp.float32),
        ],
    )()
    return out


# speedup vs baseline: 14.5380x; 1.1450x over previous
import functools
import math

import jax
import jax.numpy as jnp
from jax.experimental import pallas as pl
from jax.experimental.pallas import tpu as pltpu

_NUM_UNITS = 1024
_SCALE = float(_NUM_UNITS) ** 0.5
_BT = 256


def _pe_tile(o_ref, sinx_ref, cosx_ref, *, bt, num_units):
    t = pl.program_id(0)
    col = jax.lax.broadcasted_iota(jnp.int32, (1, num_units), 1)
    inv = jnp.exp(
        col.astype(jnp.float32) * (-2.0 * math.log(10000.0) / float(num_units))
    )

    @pl.when(t == 0)
    def _():
        phase = (col % 2).astype(jnp.float32) * (math.pi / 2.0)
        r = jax.lax.broadcasted_iota(jnp.int32, (bt, num_units), 0).astype(
            jnp.float32
        )
        x = r * inv + phase
        sinx_ref[...] = jnp.sin(x)
        cosx_ref[...] = jnp.cos(x)

    p = (t * bt).astype(jnp.float32) * inv
    sp = jnp.sin(p) * _SCALE
    cp = jnp.cos(p) * _SCALE
    val = sp * cosx_ref[...] + cp * sinx_ref[...]
    o_ref[...] = jnp.broadcast_to(val[None], o_ref.shape)

    @pl.when(t == 0)
    def _():
        o_ref[:, 0:1, :] = jnp.zeros_like(o_ref[:, 0:1, :])


def kernel(inputs):
    n, t_len = inputs.shape
    num_units = _NUM_UNITS
    bt = _BT
    grid = (t_len // bt,)
    out = pl.pallas_call(
        functools.partial(_pe_tile, bt=bt, num_units=num_units),
        grid=grid,
        out_specs=pl.BlockSpec((n, bt, num_units), lambda g: (0, g, 0)),
        out_shape=jax.ShapeDtypeStruct((n, t_len, num_units), jnp.float32),
        scratch_shapes=[
            pltpu.VMEM((bt, num_units), jnp.float32),
            pltpu.VMEM((bt, num_units), jnp.float32),
        ],
    )()
    return out
